# baseline (speedup 1.0000x reference)
import functools

import jax
import jax.numpy as jnp
from jax import lax
from jax.experimental import pallas as pl
from jax.experimental.pallas import tpu as pltpu
from jax.experimental.pallas import tpu_sc as plsc

B, S, D, E, K = 2, 4096, 4096, 64, 8
NC, NS, L = 2, 16, 16
NW = NC * NS
ROWS_PER_W = (B * S) // NW
R_CHUNK = 8
N_CHUNKS = ROWS_PER_W // R_CHUNK


@functools.partial(
    pl.kernel,
    out_type=jax.ShapeDtypeStruct((NW, D), jnp.float32),
    mesh=plsc.VectorSubcoreMesh(core_axis_name="c", subcore_axis_name="s"),
    scratch_types=[
        pltpu.VMEM((R_CHUNK, D), jnp.float32),
        pltpu.VMEM((R_CHUNK, D), jnp.float32),
        pltpu.VMEM((1, D), jnp.float32),
        pltpu.SemaphoreType.DMA,
        pltpu.SemaphoreType.DMA,
    ],
)
def _sc_pool(h_hbm, out_hbm, buf0, buf1, acc, sem0, sem1):
    wid = lax.axis_index("s") * NC + lax.axis_index("c")
    row0 = wid * ROWS_PER_W

    def zbody(j, _):
        acc[0, pl.ds(j * L, L)] = jnp.zeros((L,), jnp.float32)
        return 0

    lax.fori_loop(0, D // L, zbody, 0, unroll=8)

    bufs = (buf0, buf1)
    sems = (sem0, sem1)
    pend = [None, None]
    pend[0] = pltpu.async_copy(h_hbm.at[pl.ds(row0, R_CHUNK)], buf0, sem0)
    for c in range(N_CHUNKS):
        if c + 1 < N_CHUNKS:
            pend[(c + 1) % 2] = pltpu.async_copy(
                h_hbm.at[pl.ds(row0 + (c + 1) * R_CHUNK, R_CHUNK)],
                bufs[(c + 1) % 2], sems[(c + 1) % 2])
        pend[c % 2].wait()
        buf = bufs[c % 2]

        def abody(j, _):
            sl = pl.ds(j * L, L)
            a = acc[0, sl]
            for r in range(R_CHUNK):
                a = a + buf[r, sl]
            acc[0, sl] = a
            return 0

        lax.fori_loop(0, D // L, abody, 0, unroll=4)

    pltpu.sync_copy(acc, out_hbm.at[pl.ds(wid, 1)])


def _routing_epilogue(pooled, w, bvec, ow_ref, oi_ref, os_ref):
    logits = jax.lax.dot_general(
        pooled, w, (((1,), (1,)), ((), ())),
        preferred_element_type=jnp.float32) + bvec
    m = jnp.max(logits, axis=-1, keepdims=True)
    ex = jnp.exp(logits - m)
    sumex = jnp.sum(ex, axis=-1, keepdims=True)
    probs = ex / sumex

    iota = jax.lax.broadcasted_iota(jnp.int32, (B, E), 1)
    masked = probs
    sel_mask = jnp.zeros((B, E), jnp.float32)
    vals, idxs = [], []
    for _ in range(K):
        v = jnp.max(masked, axis=-1, keepdims=True)
        is_max = masked == v
        idx = jnp.min(jnp.where(is_max, iota, E), axis=-1,
                      keepdims=True)
        chosen = iota == idx
        sel_mask = sel_mask + chosen.astype(jnp.float32)
        vals.append(v)
        idxs.append(idx)
        masked = jnp.where(chosen, -1.0, masked)
    top_vals = jnp.concatenate(vals, axis=1)
    top_idx = jnp.concatenate(idxs, axis=1)
    top_w = top_vals / jnp.sum(top_vals, axis=-1, keepdims=True)

    f = jnp.mean(sel_mask, axis=0)
    p_mean = jnp.mean(probs, axis=0)
    lb = E * jnp.sum(f * p_mean)
    p_sq = jnp.mean(probs * probs, axis=0)
    aux = jnp.mean((p_sq - 1.0 / E) ** 2)
    lse = m[:, 0] + jnp.log(sumex[:, 0])
    z = jnp.mean(lse * lse)

    ow_ref[...] = jnp.pad(top_w, ((0, 8 - B), (0, 128 - K)))
    oi_ref[...] = jnp.pad(top_idx, ((0, 8 - B), (0, 128 - K)))
    scal = jnp.concatenate(
        [lb.reshape(1, 1), aux.reshape(1, 1), z.reshape(1, 1)], axis=1)
    os_ref[...] = jnp.pad(scal, ((0, 7), (0, 125)))


def _tc_epilogue(p_ref, w_ref, b_ref, ow_ref, oi_ref, os_ref):
    pooled = jnp.sum(p_ref[...], axis=1) * (1.0 / S)
    _routing_epilogue(pooled, w_ref[...], b_ref[...], ow_ref, oi_ref, os_ref)


@jax.jit
def kernel(hidden_states, W, b):
    h2 = hidden_states.reshape(B * S, D)
    partials = _sc_pool(h2)
    ow, oi, osc = pl.pallas_call(
        _tc_epilogue,
        out_shape=[
            jax.ShapeDtypeStruct((8, 128), jnp.float32),
            jax.ShapeDtypeStruct((8, 128), jnp.int32),
            jax.ShapeDtypeStruct((8, 128), jnp.float32),
        ],
    )(partials.reshape(B, NW // B, D), W, b.reshape(1, E))
    return (oi[:B, :K], ow[:B, :K], osc[0, 0], osc[0, 1], osc[0, 2])


# speedup vs baseline: 2.1830x; 2.1830x over previous
import functools

import jax
import jax.numpy as jnp
from jax import lax
from jax.experimental import pallas as pl
from jax.experimental.pallas import tpu as pltpu
from jax.experimental.pallas import tpu_sc as plsc

B, S, D, E, K = 2, 4096, 4096, 64, 8
NC, NS, L = 2, 16, 16
NW = NC * NS
T_SC = 1536
S_TILE = 512
N_TC_TILES = (S - T_SC) // S_TILE
W_PER_B = NW // B
ROWS_PER_W = T_SC // W_PER_B
R_CHUNK = 8
N_CHUNKS = ROWS_PER_W // R_CHUNK


@functools.partial(
    pl.kernel,
    out_type=jax.ShapeDtypeStruct((NW, D), jnp.float32),
    mesh=plsc.VectorSubcoreMesh(core_axis_name="c", subcore_axis_name="s"),
    scratch_types=[
        pltpu.VMEM((R_CHUNK, D), jnp.float32),
        pltpu.VMEM((R_CHUNK, D), jnp.float32),
        pltpu.VMEM((1, D), jnp.float32),
        pltpu.SemaphoreType.DMA,
        pltpu.SemaphoreType.DMA,
    ],
)
def _sc_pool(h_hbm, out_hbm, buf0, buf1, acc, sem0, sem1):
    wid = lax.axis_index("s") * NC + lax.axis_index("c")
    batch = wid // W_PER_B
    row0 = batch * S + (wid % W_PER_B) * ROWS_PER_W

    @plsc.parallel_loop(0, D // L, unroll=8)
    def _zero(j):
        acc[0, pl.ds(j * L, L)] = jnp.zeros((L,), jnp.float32)

    bufs = (buf0, buf1)
    sems = (sem0, sem1)
    pend = [None, None]
    pend[0] = pltpu.async_copy(h_hbm.at[pl.ds(row0, R_CHUNK)], buf0, sem0)
    for c in range(N_CHUNKS):
        if c + 1 < N_CHUNKS:
            pend[(c + 1) % 2] = pltpu.async_copy(
                h_hbm.at[pl.ds(row0 + (c + 1) * R_CHUNK, R_CHUNK)],
                bufs[(c + 1) % 2], sems[(c + 1) % 2])
        pend[c % 2].wait()
        buf = bufs[c % 2]

        @plsc.parallel_loop(0, D // L, unroll=8)
        def _accum(j):
            sl = pl.ds(j * L, L)
            a = acc[0, sl]
            for r in range(R_CHUNK):
                a = a + buf[r, sl]
            acc[0, sl] = a

    pltpu.sync_copy(acc, out_hbm.at[pl.ds(wid, 1)])


def _tc_pool_body(h_ref, out_ref):
    i = pl.program_id(0)

    @pl.when(i == 0)
    def _init():
        out_ref[...] = jnp.zeros_like(out_ref)

    out_ref[...] += jnp.sum(h_ref[...], axis=1)


def _routing_epilogue(pooled, w, bvec, ow_ref, oi_ref, os_ref):
    logits = jax.lax.dot_general(
        pooled, w, (((1,), (1,)), ((), ())),
        preferred_element_type=jnp.float32) + bvec
    m = jnp.max(logits, axis=-1, keepdims=True)
    ex = jnp.exp(logits - m)
    sumex = jnp.sum(ex, axis=-1, keepdims=True)
    probs = ex / sumex

    iota = jax.lax.broadcasted_iota(jnp.int32, (B, E), 1)
    masked = probs
    sel_mask = jnp.zeros((B, E), jnp.float32)
    vals, idxs = [], []
    for _ in range(K):
        v = jnp.max(masked, axis=-1, keepdims=True)
        is_max = masked == v
        idx = jnp.min(jnp.where(is_max, iota, E), axis=-1,
                      keepdims=True)
        chosen = iota == idx
        sel_mask = sel_mask + chosen.astype(jnp.float32)
        vals.append(v)
        idxs.append(idx)
        masked = jnp.where(chosen, -1.0, masked)
    top_vals = jnp.concatenate(vals, axis=1)
    top_idx = jnp.concatenate(idxs, axis=1)
    top_w = top_vals / jnp.sum(top_vals, axis=-1, keepdims=True)

    f = jnp.mean(sel_mask, axis=0)
    p_mean = jnp.mean(probs, axis=0)
    lb = E * jnp.sum(f * p_mean)
    p_sq = jnp.mean(probs * probs, axis=0)
    aux = jnp.mean((p_sq - 1.0 / E) ** 2)
    lse = m[:, 0] + jnp.log(sumex[:, 0])
    z = jnp.mean(lse * lse)

    ow_ref[...] = jnp.pad(top_w, ((0, 8 - B), (0, 128 - K)))
    oi_ref[...] = jnp.pad(top_idx, ((0, 8 - B), (0, 128 - K)))
    scal = jnp.concatenate(
        [lb.reshape(1, 1), aux.reshape(1, 1), z.reshape(1, 1)], axis=1)
    os_ref[...] = jnp.pad(scal, ((0, 7), (0, 125)))


def _tc_epilogue(scp_ref, tcp_ref, w_ref, b_ref, ow_ref, oi_ref, os_ref):
    pooled = (jnp.sum(scp_ref[...], axis=1) + tcp_ref[...]) * (1.0 / S)
    _routing_epilogue(pooled, w_ref[...], b_ref[...], ow_ref, oi_ref, os_ref)


@jax.jit
def kernel(hidden_states, W, b):
    h2 = hidden_states.reshape(B * S, D)
    sc_partials = _sc_pool(h2)

    tc_partial = pl.pallas_call(
        _tc_pool_body,
        grid=(N_TC_TILES,),
        in_specs=[pl.BlockSpec((B, S_TILE, D),
                               lambda i: (0, T_SC // S_TILE + i, 0))],
        out_specs=pl.BlockSpec((B, D), lambda i: (0, 0)),
        out_shape=jax.ShapeDtypeStruct((B, D), jnp.float32),
    )(hidden_states)

    ow, oi, osc = pl.pallas_call(
        _tc_epilogue,
        out_shape=[
            jax.ShapeDtypeStruct((8, 128), jnp.float32),
            jax.ShapeDtypeStruct((8, 128), jnp.int32),
            jax.ShapeDtypeStruct((8, 128), jnp.float32),
        ],
    )(sc_partials.reshape(B, W_PER_B, D), tc_partial, W, b.reshape(1, E))
    return (oi[:B, :K], ow[:B, :K], osc[0, 0], osc[0, 1], osc[0, 2])


# speedup vs baseline: 3.3896x; 1.5527x over previous
import functools

import jax
import jax.numpy as jnp
from jax.experimental import pallas as pl
from jax.experimental.pallas import tpu as pltpu

B, S, D, E, K = 2, 4096, 4096, 64, 8
S_TILE = 512
N_TILES = S // S_TILE


def _router_body(h_ref, w_ref, b_ref, oi_ref, ow_ref, lb_ref, aux_ref, z_ref,
                 acc_ref):
    i = pl.program_id(0)

    @pl.when(i == 0)
    def _init():
        acc_ref[...] = jnp.zeros_like(acc_ref)

    acc_ref[...] += jnp.sum(h_ref[...], axis=1)

    @pl.when(i == N_TILES - 1)
    def _epilogue():
        pooled = acc_ref[...] * (1.0 / S)
        logits = jax.lax.dot_general(
            pooled, w_ref[...], (((1,), (1,)), ((), ())),
            preferred_element_type=jnp.float32) + b_ref[...]
        m = jnp.max(logits, axis=-1, keepdims=True)
        ex = jnp.exp(logits - m)
        sumex = jnp.sum(ex, axis=-1, keepdims=True)
        probs = ex / sumex

        iota = jax.lax.broadcasted_iota(jnp.int32, (B, E), 1)
        masked = probs
        sel_mask = jnp.zeros((B, E), jnp.float32)
        vals, idxs = [], []
        for _ in range(K):
            v = jnp.max(masked, axis=-1, keepdims=True)
            is_max = masked == v
            idx = jnp.min(jnp.where(is_max, iota, E), axis=-1,
                          keepdims=True)
            chosen = iota == idx
            sel_mask = sel_mask + chosen.astype(jnp.float32)
            vals.append(v)
            idxs.append(idx)
            masked = jnp.where(chosen, -1.0, masked)
        top_vals = jnp.concatenate(vals, axis=1)
        top_idx = jnp.concatenate(idxs, axis=1)
        top_w = top_vals / jnp.sum(top_vals, axis=-1, keepdims=True)

        f = jnp.mean(sel_mask, axis=0)
        p_mean = jnp.mean(probs, axis=0)
        lb = E * jnp.sum(f * p_mean)
        p_sq = jnp.mean(probs * probs, axis=0)
        aux = jnp.mean((p_sq - 1.0 / E) ** 2)
        lse = m[:, 0] + jnp.log(sumex[:, 0])
        z = jnp.mean(lse * lse)

        oi_ref[...] = top_idx
        ow_ref[...] = top_w
        lb_ref[0, 0] = lb
        aux_ref[0, 0] = aux
        z_ref[0, 0] = z


@jax.jit
def kernel(hidden_states, W, b):
    oi, ow, lb, aux, z = pl.pallas_call(
        _router_body,
        grid=(N_TILES,),
        in_specs=[
            pl.BlockSpec((B, S_TILE, D), lambda i: (0, i, 0)),
            pl.BlockSpec((E, D), lambda i: (0, 0)),
            pl.BlockSpec((1, E), lambda i: (0, 0)),
        ],
        out_specs=[
            pl.BlockSpec((B, K), lambda i: (0, 0)),
            pl.BlockSpec((B, K), lambda i: (0, 0)),
            pl.BlockSpec(memory_space=pltpu.SMEM),
            pl.BlockSpec(memory_space=pltpu.SMEM),
            pl.BlockSpec(memory_space=pltpu.SMEM),
        ],
        out_shape=[
            jax.ShapeDtypeStruct((B, K), jnp.int32),
            jax.ShapeDtypeStruct((B, K), jnp.float32),
            jax.ShapeDtypeStruct((1, 1), jnp.float32),
            jax.ShapeDtypeStruct((1, 1), jnp.float32),
            jax.ShapeDtypeStruct((1, 1), jnp.float32),
        ],
        scratch_shapes=[pltpu.VMEM((B, D), jnp.float32)],
    )(hidden_states, W, b.reshape(1, E))
    return (oi, ow, lb.reshape(()), aux.reshape(()), z.reshape(()))
